# baseline (speedup 1.0000x reference)
import jax
import jax.numpy as jnp
from jax.experimental import pallas as pl
from jax.experimental.pallas import tpu as pltpu
from jax.experimental.pallas import tpu_sc as plsc

_NE = 8
_D = 2048
_FF = 4096
_JITTER = 0.01
_BF = 512
_NFB = _FF // _BF
_BH = _BF // 2
_T = 256
_SC_LANES = 16


def _router_tc_kernel(x_ref, gw_ref, logits_ref, logits_t_ref):
    xb = x_ref[...].astype(jnp.bfloat16)
    logits = jax.lax.dot_general(
        xb, gw_ref[...].astype(jnp.bfloat16), (((1,), (1,)), ((), ())),
        preferred_element_type=jnp.float32)
    logits_ref[...] = logits
    logits_t_ref[...] = logits.T


def _sc_sparsemixer_body(in_ref, out_ref):
    neg_inf = jnp.float32(-jnp.inf)
    thr = jnp.float32(2 * _JITTER)

    @pl.loop(0, 128, step=_SC_LANES)
    def _(c):
        s = [in_ref[e, pl.ds(c, _SC_LANES)] for e in range(_NE)]
        max_val = s[0]
        for e in range(1, _NE):
            max_val = jnp.maximum(max_val, s[e])
        oh1 = [s[e] >= max_val for e in range(_NE)]
        zero = jnp.zeros_like(max_val)
        ninf = zero + neg_inf

        e1 = []
        for e in range(_NE):
            mask1 = (max_val - s[e]) / jnp.maximum(jnp.abs(s[e]), max_val) > thr
            mg = jnp.where(mask1, ninf, s[e])
            e1.append(jnp.exp(mg - max_val))
        den1 = e1[0]
        for e in range(1, _NE):
            den1 = den1 + e1[e]
        mult1 = zero
        for e in range(_NE):
            mult1 = mult1 + jnp.where(oh1[e], e1[e], zero)
        mult1 = mult1 / den1

        ms = [jnp.where(oh1[e], ninf, s[e]) for e in range(_NE)]
        max_val2 = ms[0]
        for e in range(1, _NE):
            max_val2 = jnp.maximum(max_val2, ms[e])
        oh2 = [ms[e] >= max_val2 for e in range(_NE)]
        e2 = []
        for e in range(_NE):
            mask2 = ((max_val2 - s[e])
                     / jnp.maximum(jnp.abs(s[e]), max_val2) > thr)
            mg2 = jnp.where(mask2, ninf, ms[e])
            e2.append(jnp.exp(mg2 - max_val2))
        den2 = e2[0]
        for e in range(1, _NE):
            den2 = den2 + e2[e]
        mult2 = zero
        for e in range(_NE):
            mult2 = mult2 + jnp.where(oh2[e], e2[e], zero)
        mult2 = mult2 / den2

        for e in range(_NE):
            w_e = (jnp.where(oh1[e], mult1, zero)
                   + jnp.where(oh2[e], mult2, zero))
            out_ref[e, pl.ds(c, _SC_LANES)] = w_e


def _sc_sparsemixer(logits_t):
    mesh = plsc.VectorSubcoreMesh(core_axis_name="c", subcore_axis_name="s")

    @pl.kernel(out_type=jax.ShapeDtypeStruct((_NE, _T), jnp.float32),
               mesh=mesh, scratch_types=[])
    def _run(logits_hbm, w_hbm):
        pltpu.emit_pipeline(
            _sc_sparsemixer_body,
            grid=(_T // 128,),
            in_specs=[pl.BlockSpec((_NE, 128), lambda i: (0, i))],
            out_specs=[pl.BlockSpec((_NE, 128), lambda i: (0, i))],
            core_axis_name=("c", "s"),
            dimension_semantics=(pltpu.PARALLEL,),
        )(logits_hbm, w_hbm)

    return _run(logits_t)


def _moe_kernel(x_ref, wt_ref, gup_g0_ref, gup_g1_ref, gup_u0_ref,
                gup_u1_ref, dn0_ref, dn1_ref,
                out_ref, w_sc, xb_sc):
    e = pl.program_id(0)
    fb = pl.program_id(1)

    @pl.when(jnp.logical_and(e == 0, fb == 0))
    def _prep():
        xb_sc[...] = x_ref[...].astype(jnp.bfloat16)
        w_sc[...] = wt_ref[...].T

    xb = xb_sc[...]
    dn = (((1,), (1,)), ((), ()))
    lane = jax.lax.broadcasted_iota(jnp.int32, (1, _NE), 1)
    wcol = jnp.sum(jnp.where(lane == e, w_sc[...], 0.0), axis=-1,
                   keepdims=True)

    def _half(g_ref, u_ref):
        g = jax.lax.dot_general(xb, g_ref[0].astype(jnp.bfloat16), dn,
                                preferred_element_type=jnp.float32)
        u = jax.lax.dot_general(xb, u_ref[0].astype(jnp.bfloat16), dn,
                                preferred_element_type=jnp.float32)
        return g * jax.nn.sigmoid(g) * u * wcol

    hb = jnp.concatenate(
        [_half(gup_g0_ref, gup_u0_ref),
         _half(gup_g1_ref, gup_u1_ref)], axis=1).astype(jnp.bfloat16)
    y = jnp.concatenate(
        [jax.lax.dot_general(hb, dn0_ref[0].astype(jnp.bfloat16), dn,
                             preferred_element_type=jnp.float32),
         jax.lax.dot_general(hb, dn1_ref[0].astype(jnp.bfloat16), dn,
                             preferred_element_type=jnp.float32)], axis=1)

    @pl.when(jnp.logical_and(e == 0, fb == 0))
    def _init():
        out_ref[...] = y

    @pl.when(jnp.logical_or(e != 0, fb != 0))
    def _acc():
        out_ref[...] += y


def kernel(hidden_states, gate_w, gate_up_weights, down_weights):
    B, S, d = hidden_states.shape
    T = B * S
    x = hidden_states.reshape(T, d)

    logits, logits_t = pl.pallas_call(
        _router_tc_kernel,
        out_shape=[
            jax.ShapeDtypeStruct((T, _NE), jnp.float32),
            jax.ShapeDtypeStruct((_NE, T), jnp.float32),
        ],
    )(x, gate_w)

    w_t = _sc_sparsemixer(logits_t)

    out = pl.pallas_call(
        _moe_kernel,
        grid=(_NE, _NFB),
        in_specs=[
            pl.BlockSpec((T, _D), lambda e, f: (0, 0)),
            pl.BlockSpec((_NE, T), lambda e, f: (0, 0)),
            pl.BlockSpec((1, _BH, _D), lambda e, f: (e, 2 * f, 0)),
            pl.BlockSpec((1, _BH, _D), lambda e, f: (e, 2 * f + 1, 0)),
            pl.BlockSpec((1, _BH, _D),
                         lambda e, f: (e, 2 * _NFB + 2 * f, 0)),
            pl.BlockSpec((1, _BH, _D),
                         lambda e, f: (e, 2 * _NFB + 2 * f + 1, 0)),
            pl.BlockSpec((1, _D // 2, _BF), lambda e, f: (e, 0, f)),
            pl.BlockSpec((1, _D // 2, _BF), lambda e, f: (e, 1, f)),
        ],
        out_specs=pl.BlockSpec((T, _D), lambda e, f: (0, 0)),
        out_shape=jax.ShapeDtypeStruct((T, _D), jnp.float32),
        scratch_shapes=[
            pltpu.VMEM((T, _NE), jnp.float32),
            pltpu.VMEM((T, _D), jnp.bfloat16),
        ],
    )(x, w_t, gate_up_weights, gate_up_weights, gate_up_weights,
      gate_up_weights, down_weights, down_weights)

    return out.reshape(B, S, d), logits
